# baseline (speedup 1.0000x reference)
import functools

import jax
import jax.numpy as jnp
from jax import lax
from jax.experimental import pallas as pl
from jax.experimental.pallas import tpu as pltpu
from jax.experimental.pallas import tpu_sc as plsc

NUM_EMB = 1000000
DIM = 64
BATCH = 16384
SEQ = 50
NC = 2
NS = 16
NW = NC * NS
GRP = 4
NGRP = SEQ * 128 // GRP
GPW = NGRP // NW
GBLK = GRP * 8 * 128
OUT_FLAT = SEQ * 8 * 128 * 8 * 128


def _build():
    mesh = plsc.VectorSubcoreMesh(core_axis_name="c", subcore_axis_name="s")

    @functools.partial(
        pl.kernel,
        mesh=mesh,
        out_type=jax.ShapeDtypeStruct((OUT_FLAT,), jnp.float32),
        scratch_types=[
            pltpu.VMEM((2, 512), jnp.int32),
            [pltpu.VMEM((128, DIM), jnp.float32) for _ in range(GRP)],
            [pltpu.VMEM((8 * GBLK,), jnp.float32) for _ in range(2)],
            pltpu.SemaphoreType.DMA,
            [pltpu.SemaphoreType.DMA for _ in range(GRP)],
            [pltpu.SemaphoreType.DMA for _ in range(2)],
        ],
        compiler_params=pltpu.CompilerParams(
            use_tc_tiling_on_sc=False, needs_layout_passes=False
        ),
    )
    def gather_kernel(ids_hbm, table_hbm, out_hbm, ids_g, gbufs, tbufs,
                      isem, fsems, wsems):
        wid = lax.axis_index("s") * NC + lax.axis_index("c")
        gbase = wid * GPW

        iota16 = lax.iota(jnp.int32, 16)
        dvecs = [
            (iota16 + dg * 16) * 128 + ((iota16 + dg * 16) >> 3) * 3072 + ccu
            for dg in range(4)
            for ccu in range(4)
        ]

        def fire_ids(g, slot):
            gg = gbase + g
            p0 = gg * GRP
            t = p0 >> 7
            tc0 = p0 & 127
            pltpu.async_copy(
                ids_hbm.at[pl.ds(t * BATCH + tc0 * 128, 512)],
                ids_g.at[slot],
                isem,
            )

        def wait_ids():
            pltpu.make_async_copy(
                ids_hbm.at[pl.ds(0, 512)], ids_g.at[0], isem
            ).wait()

        def fill(g, slot, j):
            pltpu.async_copy(
                table_hbm.at[ids_g.at[slot, pl.ds(j * 128, 128)]],
                gbufs[j],
                fsems[j],
            )

        def wait_fill(j):
            pltpu.make_async_copy(
                table_hbm.at[pl.ds(0, 128)], gbufs[j], fsems[j]
            ).wait()

        def transpose(j, ts):
            def body(i, vecs):
                cc4 = i * 4
                out = []
                for dg in range(4):
                    for ccu in range(4):
                        q = dg * 4 + ccu
                        v = gbufs[j][cc4 + ccu, pl.ds(dg * 16, 16)]
                        plsc.store_scatter(tbufs[ts], [vecs[q]], v)
                        out.append(vecs[q] + 4)
                return tuple(out)

            lax.fori_loop(0, 32, body, tuple(v + j * 1024 for v in dvecs))

        def drain(g, ts):
            gg = gbase + g
            p0 = gg * GRP
            t = p0 >> 7
            tc0 = p0 & 127
            for tr in range(8):
                pltpu.async_copy(
                    tbufs[ts].at[pl.ds(tr * GBLK, GBLK)],
                    out_hbm.at[
                        pl.ds((t * 8 + tr) * 131072 + tc0 * 1024, GBLK)
                    ],
                    wsems[ts],
                )

        def wait_drain(ts):
            pltpu.make_async_copy(
                tbufs[ts], out_hbm.at[pl.ds(0, 8 * GBLK)], wsems[ts]
            ).wait()

        fire_ids(0, 0)
        wait_ids()
        fire_ids(1, 1)
        for j in range(GRP):
            fill(0, 0, j)

        def lap(g, gslot, ts, do_wait_drain, do_fill, prefetch):
            if prefetch:
                wait_ids()
                fire_ids(jnp.minimum(g + 2, GPW - 1), gslot)
            if do_wait_drain:
                wait_drain(ts)
            for j in range(GRP):
                wait_fill(j)
                transpose(j, ts)
                if do_fill:
                    fill(g + 1, 1 - gslot, j)
            drain(g, ts)

        lap(0, 0, 0, do_wait_drain=False, do_fill=True, prefetch=True)
        lap(1, 1, 1, do_wait_drain=False, do_fill=True, prefetch=True)

        def outer(m, carry):
            g = 2 * m
            lap(g, 0, 0, do_wait_drain=True, do_fill=True, prefetch=True)
            lap(g + 1, 1, 1, do_wait_drain=True, do_fill=True, prefetch=True)
            return carry

        lax.fori_loop(1, (GPW - 2) // 2, outer, 0)

        lap(GPW - 2, 0, 0, do_wait_drain=True, do_fill=True, prefetch=True)
        lap(GPW - 1, 1, 1, do_wait_drain=True, do_fill=False, prefetch=False)
        wait_ids()
        for ts in range(2):
            wait_drain(ts)

    return gather_kernel


_gather = _build()


def kernel(token_ids, EmbeddingLayer):
    ids = token_ids.astype(jnp.int32).T.reshape(-1)
    x = _gather(ids, EmbeddingLayer)
    x5 = x.reshape(SEQ, 8, 128, 8, 128)
    return x5.transpose(2, 4, 0, 1, 3).reshape(BATCH, SEQ, DIM)


# speedup vs baseline: 1.0048x; 1.0048x over previous
import functools

import jax
import jax.numpy as jnp
from jax import lax
from jax.experimental import pallas as pl
from jax.experimental.pallas import tpu as pltpu
from jax.experimental.pallas import tpu_sc as plsc

NUM_EMB = 1000000
DIM = 64
BATCH = 16384
SEQ = 50
NC = 2
NS = 16
NW = NC * NS
GRP = 4
NGRP = SEQ * 128 // GRP
GPW = NGRP // NW
GBLK = GRP * 8 * 128
OUT_FLAT = SEQ * 8 * 128 * 8 * 128


def _build():
    mesh = plsc.VectorSubcoreMesh(core_axis_name="c", subcore_axis_name="s")

    @functools.partial(
        pl.kernel,
        mesh=mesh,
        out_type=jax.ShapeDtypeStruct((OUT_FLAT,), jnp.float32),
        scratch_types=[
            pltpu.VMEM((2 * GRP, 128), jnp.int32),
            [pltpu.VMEM((128, DIM), jnp.float32) for _ in range(GRP)],
            [pltpu.VMEM((8 * GBLK,), jnp.float32) for _ in range(2)],
            pltpu.SemaphoreType.DMA,
            [pltpu.SemaphoreType.DMA for _ in range(GRP)],
            [pltpu.SemaphoreType.DMA for _ in range(2)],
        ],
        compiler_params=pltpu.CompilerParams(
            use_tc_tiling_on_sc=False, needs_layout_passes=False
        ),
    )
    def gather_kernel(ids_hbm, table_hbm, out_hbm, ids_g, gbufs, tbufs,
                      isem, fsems, wsems):
        wid = lax.axis_index("s") * NC + lax.axis_index("c")
        gbase = wid * GPW

        iota16 = lax.iota(jnp.int32, 16)
        dvecs = [
            (iota16 + dg * 16) * 128 + ((iota16 + dg * 16) >> 3) * 3072 + ccu
            for dg in range(4)
            for ccu in range(4)
        ]

        def fire_ids(g, slot):
            gg = gbase + g
            pltpu.async_copy(
                ids_hbm.at[pl.ds(gg * GRP, GRP)],
                ids_g.at[pl.ds(slot * GRP, GRP)],
                isem,
            )

        def wait_ids():
            pltpu.make_async_copy(
                ids_hbm.at[pl.ds(0, GRP)], ids_g.at[pl.ds(0, GRP)], isem
            ).wait()

        def fill(g, slot, j):
            pltpu.async_copy(
                table_hbm.at[ids_g.at[slot * GRP + j]],
                gbufs[j],
                fsems[j],
            )

        def wait_fill(j):
            pltpu.make_async_copy(
                table_hbm.at[pl.ds(0, 128)], gbufs[j], fsems[j]
            ).wait()

        def transpose(j, ts):
            def body(i, vecs):
                cc4 = i * 4
                out = []
                for dg in range(4):
                    for ccu in range(4):
                        q = dg * 4 + ccu
                        v = gbufs[j][cc4 + ccu, pl.ds(dg * 16, 16)]
                        plsc.store_scatter(tbufs[ts], [vecs[q]], v)
                        out.append(vecs[q] + 4)
                return tuple(out)

            lax.fori_loop(0, 32, body, tuple(v + j * 1024 for v in dvecs))

        def drain(g, ts):
            gg = gbase + g
            p0 = gg * GRP
            t = p0 >> 7
            tc0 = p0 & 127
            for tr in range(8):
                pltpu.async_copy(
                    tbufs[ts].at[pl.ds(tr * GBLK, GBLK)],
                    out_hbm.at[
                        pl.ds((t * 8 + tr) * 131072 + tc0 * 1024, GBLK)
                    ],
                    wsems[ts],
                )

        def wait_drain(ts):
            pltpu.make_async_copy(
                tbufs[ts], out_hbm.at[pl.ds(0, 8 * GBLK)], wsems[ts]
            ).wait()

        fire_ids(0, 0)
        wait_ids()
        fire_ids(1, 1)
        for j in range(GRP):
            fill(0, 0, j)

        def lap(g, gslot, ts, do_wait_drain, do_fill, prefetch):
            if prefetch:
                wait_ids()
                fire_ids(jnp.minimum(g + 2, GPW - 1), gslot)
            if do_wait_drain:
                wait_drain(ts)
            for j in range(GRP):
                wait_fill(j)
                transpose(j, ts)
                if do_fill:
                    fill(g + 1, 1 - gslot, j)
            drain(g, ts)

        lap(0, 0, 0, do_wait_drain=False, do_fill=True, prefetch=True)
        lap(1, 1, 1, do_wait_drain=False, do_fill=True, prefetch=True)

        def outer(m, carry):
            g = 2 * m
            lap(g, 0, 0, do_wait_drain=True, do_fill=True, prefetch=True)
            lap(g + 1, 1, 1, do_wait_drain=True, do_fill=True, prefetch=True)
            return carry

        lax.fori_loop(1, (GPW - 2) // 2, outer, 0)

        lap(GPW - 2, 0, 0, do_wait_drain=True, do_fill=True, prefetch=True)
        lap(GPW - 1, 1, 1, do_wait_drain=True, do_fill=False, prefetch=False)
        wait_ids()
        for ts in range(2):
            wait_drain(ts)

    return gather_kernel


_gather = _build()


def kernel(token_ids, EmbeddingLayer):
    ids = token_ids.astype(jnp.int32).T.reshape(SEQ * 128, 128)
    x = _gather(ids, EmbeddingLayer)
    x5 = x.reshape(SEQ, 8, 128, 8, 128)
    return x5.transpose(2, 4, 0, 1, 3).reshape(BATCH, SEQ, DIM)


# speedup vs baseline: 1.5553x; 1.5479x over previous
import functools

import jax
import jax.numpy as jnp
from jax import lax
from jax.experimental import pallas as pl
from jax.experimental.pallas import tpu as pltpu
from jax.experimental.pallas import tpu_sc as plsc

NUM_EMB = 1000000
DIM = 64
BATCH = 16384
SEQ = 50
NC = 2
NS = 16
NW = NC * NS
GRP = 4
NGRP = SEQ * 128 // GRP
GPW = NGRP // NW
GBLK = GRP * 8 * 128
OUT_FLAT = SEQ * 8 * 128 * 8 * 128


def _build():
    mesh = plsc.VectorSubcoreMesh(core_axis_name="c", subcore_axis_name="s")

    @functools.partial(
        pl.kernel,
        mesh=mesh,
        out_type=jax.ShapeDtypeStruct((OUT_FLAT,), jnp.float32),
        scratch_types=[
            pltpu.VMEM((2 * GRP, 128), jnp.int32),
            [pltpu.VMEM((128, DIM), jnp.float32) for _ in range(GRP)],
            [pltpu.VMEM((8 * GBLK,), jnp.float32) for _ in range(2)],
            pltpu.SemaphoreType.DMA,
            [pltpu.SemaphoreType.DMA for _ in range(GRP)],
            [pltpu.SemaphoreType.DMA for _ in range(2)],
        ],
        compiler_params=pltpu.CompilerParams(
            use_tc_tiling_on_sc=False, needs_layout_passes=False
        ),
    )
    def gather_kernel(ids_hbm, table_hbm, out_hbm, ids_g, gbufs, tbufs,
                      isem, fsems, wsems):
        wid = lax.axis_index("s") * NC + lax.axis_index("c")
        gbase = wid * GPW

        iota16 = lax.iota(jnp.int32, 16)
        ccrows = [iota16 + g8 * 16 for g8 in range(8)]

        def fire_ids(g, slot):
            gg = gbase + g
            pltpu.async_copy(
                ids_hbm.at[pl.ds(gg * GRP, GRP)],
                ids_g.at[pl.ds(slot * GRP, GRP)],
                isem,
            )

        def wait_ids():
            pltpu.make_async_copy(
                ids_hbm.at[pl.ds(0, GRP)], ids_g.at[pl.ds(0, GRP)], isem
            ).wait()

        def fill(g, slot, j):
            pltpu.async_copy(
                table_hbm.at[ids_g.at[slot * GRP + j]],
                gbufs[j],
                fsems[j],
            )

        def wait_fill(j):
            pltpu.make_async_copy(
                table_hbm.at[pl.ds(0, 128)], gbufs[j], fsems[j]
            ).wait()

        def transpose(j, ts):
            ccvecs = [iota16 + (g8 * 16 + j * 1024) for g8 in range(8)]

            def body(d0, carry):
                diag = (iota16 + d0) & 63
                tbase = (diag >> 3) * 3072 + diag * 128
                for g8 in range(8):
                    v = plsc.load_gather(gbufs[j], [ccrows[g8], diag])
                    plsc.store_scatter(tbufs[ts], [tbase + ccvecs[g8]], v)
                return carry

            lax.fori_loop(0, DIM, body, 0)

        def drain(g, ts):
            gg = gbase + g
            p0 = gg * GRP
            t = p0 >> 7
            tc0 = p0 & 127
            for tr in range(8):
                pltpu.async_copy(
                    tbufs[ts].at[pl.ds(tr * GBLK, GBLK)],
                    out_hbm.at[
                        pl.ds((t * 8 + tr) * 131072 + tc0 * 1024, GBLK)
                    ],
                    wsems[ts],
                )

        def wait_drain(ts):
            pltpu.make_async_copy(
                tbufs[ts], out_hbm.at[pl.ds(0, 8 * GBLK)], wsems[ts]
            ).wait()

        fire_ids(0, 0)
        wait_ids()
        fire_ids(1, 1)
        for j in range(GRP):
            fill(0, 0, j)

        def lap(g, gslot, ts, do_wait_drain, do_fill, prefetch):
            if prefetch:
                wait_ids()
                fire_ids(jnp.minimum(g + 2, GPW - 1), gslot)
            if do_wait_drain:
                wait_drain(ts)
            for j in range(GRP):
                wait_fill(j)
                transpose(j, ts)
                if do_fill:
                    fill(g + 1, 1 - gslot, j)
            drain(g, ts)

        lap(0, 0, 0, do_wait_drain=False, do_fill=True, prefetch=True)
        lap(1, 1, 1, do_wait_drain=False, do_fill=True, prefetch=True)

        def outer(m, carry):
            g = 2 * m
            lap(g, 0, 0, do_wait_drain=True, do_fill=True, prefetch=True)
            lap(g + 1, 1, 1, do_wait_drain=True, do_fill=True, prefetch=True)
            return carry

        lax.fori_loop(1, (GPW - 2) // 2, outer, 0)

        lap(GPW - 2, 0, 0, do_wait_drain=True, do_fill=True, prefetch=True)
        lap(GPW - 1, 1, 1, do_wait_drain=True, do_fill=False, prefetch=False)
        wait_ids()
        for ts in range(2):
            wait_drain(ts)

    return gather_kernel


_gather = _build()


def kernel(token_ids, EmbeddingLayer):
    ids = token_ids.astype(jnp.int32).T.reshape(SEQ * 128, 128)
    x = _gather(ids, EmbeddingLayer)
    x5 = x.reshape(SEQ, 8, 128, 8, 128)
    return x5.transpose(2, 4, 0, 1, 3).reshape(BATCH, SEQ, DIM)
